# Initial kernel scaffold; baseline (speedup 1.0000x reference)
import jax
import jax.numpy as jnp
from jax.experimental import pallas as pl


def kernel(x, W_router, W_up, W_down):
    raise NotImplementedError("write your pallas kernel here")


# speedup vs baseline: 1.1612x; 1.1612x over previous
import functools

import jax
import jax.numpy as jnp
from jax.experimental import pallas as pl
from jax.experimental.pallas import tpu as pltpu

S, DIM, HID, E, TOPK = 2048, 768, 3072, 8, 2
HCHUNK = 1024


def _moe_dense_body(x_ref, wr_ref, wup_ref, wdn_ref, out_ref, w_sc, xb_sc):
    e = pl.program_id(0)
    hc = pl.program_id(1)
    first = (e == 0) & (hc == 0)

    @pl.when(first)
    def _router():
        logits = jnp.dot(
            x_ref[...], wr_ref[...], preferred_element_type=jnp.float32
        )
        p = jax.nn.softmax(logits, axis=-1)
        col = jax.lax.broadcasted_iota(jnp.int32, p.shape, 1)
        m1 = jnp.max(p, axis=-1, keepdims=True)
        i1 = jnp.min(jnp.where(p == m1, col, E), axis=-1, keepdims=True)
        p_rest = jnp.where(col == i1, -jnp.inf, p)
        m2 = jnp.max(p_rest, axis=-1, keepdims=True)
        i2 = jnp.min(jnp.where(p_rest == m2, col, E), axis=-1, keepdims=True)
        mask = (col == i1) | (col == i2)
        w_sc[...] = jnp.where(mask, p, 0.0) / (m1 + m2)
        xb_sc[...] = x_ref[...].astype(jnp.bfloat16)

    col = jax.lax.broadcasted_iota(jnp.int32, (S, E), 1)
    w_col = jnp.sum(jnp.where(col == e, w_sc[...], 0.0), axis=-1, keepdims=True)

    h = jnp.dot(xb_sc[...], wup_ref[0], preferred_element_type=jnp.float32)
    h = jax.nn.gelu(h)
    y = jnp.dot(
        h.astype(jnp.bfloat16), wdn_ref[0], preferred_element_type=jnp.float32
    )
    acc = w_col * y

    @pl.when(first)
    def _init():
        out_ref[...] = acc

    @pl.when(jnp.logical_not(first))
    def _acc():
        out_ref[...] += acc


@jax.jit
def _moe(x2d, W_router, W_up_bf, W_dn_bf):
    return pl.pallas_call(
        _moe_dense_body,
        grid=(E, HID // HCHUNK),
        in_specs=[
            pl.BlockSpec((S, DIM), lambda e, hc: (0, 0)),
            pl.BlockSpec((DIM, E), lambda e, hc: (0, 0)),
            pl.BlockSpec((1, DIM, HCHUNK), lambda e, hc: (e, 0, hc)),
            pl.BlockSpec((1, HCHUNK, DIM), lambda e, hc: (e, hc, 0)),
        ],
        out_specs=pl.BlockSpec((S, DIM), lambda e, hc: (0, 0)),
        out_shape=jax.ShapeDtypeStruct((S, DIM), jnp.float32),
        scratch_shapes=[
            pltpu.VMEM((S, E), jnp.float32),
            pltpu.VMEM((S, DIM), jnp.bfloat16),
        ],
    )(x2d, W_router, W_up_bf, W_dn_bf)


def kernel(x, W_router, W_up, W_down):
    x2d = x.reshape(S, DIM)
    out = _moe(
        x2d,
        W_router,
        W_up.astype(jnp.bfloat16),
        W_down.astype(jnp.bfloat16),
    )
    return out.reshape(x.shape)
